# baseline (speedup 1.0000x reference)
import functools

import jax
import jax.numpy as jnp
from jax import lax
from jax.experimental import pallas as pl
from jax.experimental.pallas import tpu as pltpu
from jax.experimental.pallas import tpu_sc as plsc

VOCAB = 1000000
EMBED = 64
HIDDEN = 256
CLASSES = 100
BATCH = 4096
SEQ = 200

NC = 2
NS = 16
NW = NC * NS
LANES = 16

CHUNK = SEQ // 2
ROWS_W = BATCH // NW
NCH = ROWS_W * 2
NBUF = 4
NG = NCH // NBUF
NVEC = EMBED // LANES


def _sc_pool_body(tokens_hbm, table_hbm, out_hbm, idx_v, bufs_v, out_v,
                  s0, s1, s2, s3):
    sems = (s0, s1, s2, s3)
    wid = lax.axis_index("s") * NC + lax.axis_index("c")

    pltpu.sync_copy(tokens_hbm.at[pl.ds(wid * NCH, NCH)], idx_v)

    for b in range(NBUF):
        pltpu.make_async_copy(
            table_hbm.at[idx_v.at[b]], bufs_v.at[b], sems[b]).start()

    def make_inner(b):
        def inner(j, acc):
            return tuple(acc[d] + bufs_v[b, j, pl.ds(LANES * d, LANES)]
                         for d in range(NVEC))
        return inner

    zero = jnp.zeros((LANES,), jnp.float32)
    inv_seq = jnp.float32(1.0 / SEQ)

    def body(g, carry):
        for p in range(NBUF // 2):
            row = 2 * g + p
            acc = (zero,) * NVEC
            for h in range(2):
                b = 2 * p + h
                c = g * NBUF + b
                pltpu.make_async_copy(
                    table_hbm.at[idx_v.at[c]], bufs_v.at[b], sems[b]).wait()
                acc = lax.fori_loop(0, CHUNK, make_inner(b), acc, unroll=4)

                @pl.when(g < NG - 1)
                def _():
                    pltpu.make_async_copy(
                        table_hbm.at[idx_v.at[c + NBUF]],
                        bufs_v.at[b], sems[b]).start()

            for d in range(NVEC):
                out_v[row, pl.ds(LANES * d, LANES)] = acc[d] * inv_seq
        return carry

    lax.fori_loop(0, NG, body, 0)

    pltpu.sync_copy(out_v, out_hbm.at[pl.ds(wid * ROWS_W, ROWS_W)])


_sc_pool = functools.partial(
    pl.kernel,
    out_type=jax.ShapeDtypeStruct((BATCH, EMBED), jnp.float32),
    mesh=plsc.VectorSubcoreMesh(core_axis_name="c", subcore_axis_name="s",
                                num_cores=NC, num_subcores=NS),
    scratch_types=[
        pltpu.VMEM((NCH, CHUNK), jnp.int32),
        pltpu.VMEM((NBUF, CHUNK, EMBED), jnp.float32),
        pltpu.VMEM((ROWS_W, EMBED), jnp.float32),
        pltpu.SemaphoreType.DMA,
        pltpu.SemaphoreType.DMA,
        pltpu.SemaphoreType.DMA,
        pltpu.SemaphoreType.DMA,
    ],
    compiler_params=pltpu.CompilerParams(use_tc_tiling_on_sc=False),
)(_sc_pool_body)


def _mlp_body(x_ref, W1_ref, b1_ref, gamma_ref, beta_ref, W2_ref, b2_ref,
              o_ref):
    x = x_ref[...]
    h = jnp.dot(x, W1_ref[...], preferred_element_type=jnp.float32,
                precision=lax.Precision.HIGHEST) + b1_ref[...]
    mu = jnp.mean(h, axis=0, keepdims=True)
    dev = h - mu
    var = jnp.mean(dev * dev, axis=0, keepdims=True)
    h = dev * lax.rsqrt(var + 1e-5) * gamma_ref[...] + beta_ref[...]
    h = jnp.maximum(h, 0.0)
    logits = jnp.dot(h, W2_ref[...], preferred_element_type=jnp.float32,
                     precision=lax.Precision.HIGHEST) + b2_ref[...]
    m = jnp.max(logits, axis=1, keepdims=True)
    e = jnp.exp(logits - m)
    o_ref[...] = e / jnp.sum(e, axis=1, keepdims=True)


_mlp = pl.pallas_call(
    _mlp_body,
    out_shape=jax.ShapeDtypeStruct((BATCH, CLASSES), jnp.float32),
)


def kernel(tokens, table, W1, b1, gamma, beta, W2, b2):
    tokens2 = tokens.reshape(BATCH * 2, CHUNK)
    pooled = _sc_pool(tokens2, table)
    return _mlp(pooled, W1, b1.reshape(1, HIDDEN), gamma.reshape(1, HIDDEN),
                beta.reshape(1, HIDDEN), W2, b2.reshape(1, CLASSES))


# speedup vs baseline: 1.0004x; 1.0004x over previous
import functools

import jax
import jax.numpy as jnp
from jax import lax
from jax.experimental import pallas as pl
from jax.experimental.pallas import tpu as pltpu
from jax.experimental.pallas import tpu_sc as plsc

VOCAB = 1000000
EMBED = 64
HIDDEN = 256
CLASSES = 100
BATCH = 4096
SEQ = 200

NC = 2
NS = 16
NW = NC * NS
LANES = 16

CH = (104, 96)
CHOFF = (0, 104)
CHMAX = 104
ROWS_W = BATCH // NW
NBUF = 4
NG = ROWS_W // 2
NVEC = EMBED // LANES


def _sc_pool_body(tokens_hbm, table_hbm, out_hbm, idx_v, bufs_v, out_v,
                  s0, s1, s2, s3):
    sems = (s0, s1, s2, s3)
    wid = lax.axis_index("s") * NC + lax.axis_index("c")

    pltpu.sync_copy(tokens_hbm.at[pl.ds(wid * ROWS_W, ROWS_W)], idx_v)

    def idx_slice(row, h):
        return idx_v.at[row, pl.ds(CHOFF[h], CH[h])]

    for p in range(2):
        for h in range(2):
            b = 2 * p + h
            pltpu.make_async_copy(
                table_hbm.at[idx_slice(p, h)],
                bufs_v.at[b].at[pl.ds(0, CH[h])], sems[b]).start()

    def make_inner(b):
        def inner(j, acc):
            return tuple(acc[d] + bufs_v[b, j, pl.ds(LANES * d, LANES)]
                         for d in range(NVEC))
        return inner

    zero = jnp.zeros((LANES,), jnp.float32)
    inv_seq = jnp.float32(1.0 / SEQ)

    def body(g, carry):
        for p in range(2):
            row = 2 * g + p
            acc = (zero,) * NVEC
            for h in range(2):
                b = 2 * p + h
                pltpu.make_async_copy(
                    table_hbm.at[idx_slice(row, h)],
                    bufs_v.at[b].at[pl.ds(0, CH[h])], sems[b]).wait()
                acc = lax.fori_loop(0, CH[h], make_inner(b), acc, unroll=4)

                @pl.when(g < NG - 1)
                def _():
                    pltpu.make_async_copy(
                        table_hbm.at[idx_slice(row + 2, h)],
                        bufs_v.at[b].at[pl.ds(0, CH[h])], sems[b]).start()

            for d in range(NVEC):
                out_v[row, pl.ds(LANES * d, LANES)] = acc[d] * inv_seq
        return carry

    lax.fori_loop(0, NG, body, 0)

    pltpu.sync_copy(out_v, out_hbm.at[pl.ds(wid * ROWS_W, ROWS_W)])


_sc_pool = functools.partial(
    pl.kernel,
    out_type=jax.ShapeDtypeStruct((BATCH, EMBED), jnp.float32),
    mesh=plsc.VectorSubcoreMesh(core_axis_name="c", subcore_axis_name="s",
                                num_cores=NC, num_subcores=NS),
    scratch_types=[
        pltpu.VMEM((ROWS_W, SEQ), jnp.int32),
        pltpu.VMEM((NBUF, CHMAX, EMBED), jnp.float32),
        pltpu.VMEM((ROWS_W, EMBED), jnp.float32),
        pltpu.SemaphoreType.DMA,
        pltpu.SemaphoreType.DMA,
        pltpu.SemaphoreType.DMA,
        pltpu.SemaphoreType.DMA,
    ],
    compiler_params=pltpu.CompilerParams(use_tc_tiling_on_sc=False),
)(_sc_pool_body)


def _mlp_body(x_ref, W1_ref, b1_ref, gamma_ref, beta_ref, W2_ref, b2_ref,
              o_ref):
    x = x_ref[...]
    h = jnp.dot(x, W1_ref[...], preferred_element_type=jnp.float32,
                precision=lax.Precision.HIGHEST) + b1_ref[...]
    mu = jnp.mean(h, axis=0, keepdims=True)
    dev = h - mu
    var = jnp.mean(dev * dev, axis=0, keepdims=True)
    h = dev * lax.rsqrt(var + 1e-5) * gamma_ref[...] + beta_ref[...]
    h = jnp.maximum(h, 0.0)
    logits = jnp.dot(h, W2_ref[...], preferred_element_type=jnp.float32,
                     precision=lax.Precision.HIGHEST) + b2_ref[...]
    m = jnp.max(logits, axis=1, keepdims=True)
    e = jnp.exp(logits - m)
    o_ref[...] = e / jnp.sum(e, axis=1, keepdims=True)


_mlp = pl.pallas_call(
    _mlp_body,
    out_shape=jax.ShapeDtypeStruct((BATCH, CLASSES), jnp.float32),
)


def kernel(tokens, table, W1, b1, gamma, beta, W2, b2):
    pooled = _sc_pool(tokens, table)
    return _mlp(pooled, W1, b1.reshape(1, HIDDEN), gamma.reshape(1, HIDDEN),
                beta.reshape(1, HIDDEN), W2, b2.reshape(1, CLASSES))


# speedup vs baseline: 1.8392x; 1.8384x over previous
import functools

import jax
import jax.numpy as jnp
from jax import lax
from jax.experimental import pallas as pl
from jax.experimental.pallas import tpu as pltpu
from jax.experimental.pallas import tpu_sc as plsc

VOCAB = 1000000
EMBED = 64
HIDDEN = 256
CLASSES = 100
BATCH = 4096
SEQ = 200

NC = 2
NS = 16
NW = NC * NS
LANES = 16

CH = (104, 96)
CHOFF = (0, 104)
CHMAX = 104
ROWS_W = BATCH // NW
NBUF = 4
NG = ROWS_W // 2
NVEC = EMBED // LANES


def _sc_pool_body(tokens_hbm, table_hbm, out_hbm, idx_v, bufs_v, out_v,
                  s0, s1, s2, s3):
    sems = (s0, s1, s2, s3)
    wid = lax.axis_index("s") * NC + lax.axis_index("c")

    pltpu.sync_copy(tokens_hbm.at[pl.ds(wid * ROWS_W, ROWS_W)],
                    idx_v.at[:, pl.ds(0, SEQ)])

    def remap(r_, carry):
        for k in range(13):
            t = idx_v[r_, pl.ds(LANES * k, LANES)]
            row = (((t >> 13) << 13) + ((t & 4095) << 1)
                   + ((t >> 12) & 1))
            idx_v[r_, pl.ds(LANES * k, LANES)] = row
        return carry

    lax.fori_loop(0, ROWS_W, remap, 0)

    def idx_slice(row, h):
        return idx_v.at[row, pl.ds(CHOFF[h], CH[h])]

    for p in range(2):
        for h in range(2):
            b = 2 * p + h
            pltpu.make_async_copy(
                table_hbm.at[idx_slice(p, h)],
                bufs_v.at[b].at[pl.ds(0, CH[h])], sems[b]).start()

    def make_inner(b):
        def inner(j, acc):
            return tuple(acc[d] + bufs_v[b, j, pl.ds(LANES * d, LANES)]
                         for d in range(NVEC))
        return inner

    zero = jnp.zeros((LANES,), jnp.float32)
    inv_seq = jnp.float32(1.0 / SEQ)

    def body(g, carry):
        for p in range(2):
            row = 2 * g + p
            acc = (zero,) * NVEC
            for h in range(2):
                b = 2 * p + h
                pltpu.make_async_copy(
                    table_hbm.at[idx_slice(row, h)],
                    bufs_v.at[b].at[pl.ds(0, CH[h])], sems[b]).wait()
                acc = lax.fori_loop(0, CH[h], make_inner(b), acc, unroll=4)

                @pl.when(g < NG - 1)
                def _():
                    pltpu.make_async_copy(
                        table_hbm.at[idx_slice(row + 2, h)],
                        bufs_v.at[b].at[pl.ds(0, CH[h])], sems[b]).start()

            for d in range(NVEC):
                out_v[row, pl.ds(LANES * d, LANES)] = acc[d] * inv_seq
        return carry

    lax.fori_loop(0, NG, body, 0)

    pltpu.sync_copy(out_v, out_hbm.at[pl.ds(wid * ROWS_W, ROWS_W)])


_sc_pool = functools.partial(
    pl.kernel,
    out_type=jax.ShapeDtypeStruct((BATCH, EMBED), jnp.float32),
    mesh=plsc.VectorSubcoreMesh(core_axis_name="c", subcore_axis_name="s",
                                num_cores=NC, num_subcores=NS),
    scratch_types=[
        pltpu.VMEM((ROWS_W, 208), jnp.int32),
        pltpu.VMEM((NBUF, CHMAX, EMBED), jnp.float32),
        pltpu.VMEM((ROWS_W, EMBED), jnp.float32),
        pltpu.SemaphoreType.DMA,
        pltpu.SemaphoreType.DMA,
        pltpu.SemaphoreType.DMA,
        pltpu.SemaphoreType.DMA,
    ],
    compiler_params=pltpu.CompilerParams(use_tc_tiling_on_sc=False),
)(_sc_pool_body)


VCH = 8192
NBLK = (VOCAB + VCH - 1) // VCH


HALF = VCH // 2
VPAD = NBLK * VCH


def _tc_repack_body(tT_ref, o_ref):
    x = tT_ref[...]
    o_ref[:, 0:EMBED] = x[:, 0:HALF].T
    o_ref[:, EMBED:2 * EMBED] = x[:, HALF:VCH].T


_tc_repack = pl.pallas_call(
    _tc_repack_body,
    grid=(NBLK,),
    in_specs=[pl.BlockSpec((EMBED, VCH), lambda i: (0, i))],
    out_specs=pl.BlockSpec((HALF, 2 * EMBED), lambda i: (i, 0)),
    out_shape=jax.ShapeDtypeStruct((NBLK * HALF, 2 * EMBED), jnp.float32),
)


def _mlp_body(x_ref, W1_ref, b1_ref, gamma_ref, beta_ref, W2_ref, b2_ref,
              o_ref):
    x = x_ref[...]
    h = jnp.dot(x, W1_ref[...], preferred_element_type=jnp.float32,
                precision=lax.Precision.HIGHEST) + b1_ref[...]
    mu = jnp.mean(h, axis=0, keepdims=True)
    dev = h - mu
    var = jnp.mean(dev * dev, axis=0, keepdims=True)
    h = dev * lax.rsqrt(var + 1e-5) * gamma_ref[...] + beta_ref[...]
    h = jnp.maximum(h, 0.0)
    logits = jnp.dot(h, W2_ref[...], preferred_element_type=jnp.float32,
                     precision=lax.Precision.HIGHEST) + b2_ref[...]
    m = jnp.max(logits, axis=1, keepdims=True)
    e = jnp.exp(logits - m)
    o_ref[...] = e / jnp.sum(e, axis=1, keepdims=True)


_mlp = pl.pallas_call(
    _mlp_body,
    out_shape=jax.ShapeDtypeStruct((BATCH, CLASSES), jnp.float32),
)


def kernel(tokens, table, W1, b1, gamma, beta, W2, b2):
    packed = _tc_repack(table.T)
    pooled = _sc_pool(tokens, packed.reshape(VPAD, EMBED))
    return _mlp(pooled, W1, b1.reshape(1, HIDDEN), gamma.reshape(1, HIDDEN),
                beta.reshape(1, HIDDEN), W2, b2.reshape(1, CLASSES))


# speedup vs baseline: 2.0044x; 1.0898x over previous
import functools

import jax
import jax.numpy as jnp
from jax import lax
from jax.experimental import pallas as pl
from jax.experimental.pallas import tpu as pltpu
from jax.experimental.pallas import tpu_sc as plsc

VOCAB = 1000000
EMBED = 64
HIDDEN = 256
CLASSES = 100
BATCH = 4096
SEQ = 200

NC = 2
NS = 16
NW = NC * NS
LANES = 16

CH = (104, 96)
CHOFF = (0, 104)
CHMAX = 104
VSH = 14
ROWS_W = BATCH // NW
NBUF = 4
NG = ROWS_W // 2
NVEC = EMBED // LANES


def _sc_pool_body(tokens_hbm, table_hbm, out_hbm, idx_v, bufs_v, out_v,
                  s0, s1, s2, s3):
    sems = (s0, s1, s2, s3)
    wid = lax.axis_index("s") * NC + lax.axis_index("c")

    pltpu.sync_copy(tokens_hbm.at[pl.ds(wid * ROWS_W, ROWS_W)],
                    idx_v.at[:, pl.ds(0, SEQ)])

    def remap(r_, carry):
        for k in range(13):
            t = idx_v[r_, pl.ds(LANES * k, LANES)]
            row = (((t >> VSH) << VSH) + ((t & (VCH // 2 - 1)) << 1)
                   + ((t >> (VSH - 1)) & 1))
            idx_v[r_, pl.ds(LANES * k, LANES)] = row
        return carry

    lax.fori_loop(0, ROWS_W, remap, 0)

    def idx_slice(row, h):
        return idx_v.at[row, pl.ds(CHOFF[h], CH[h])]

    for p in range(2):
        for h in range(2):
            b = 2 * p + h
            pltpu.make_async_copy(
                table_hbm.at[idx_slice(p, h)],
                bufs_v.at[b].at[pl.ds(0, CH[h])], sems[b]).start()

    def make_inner(b):
        def inner(j, acc):
            return tuple(acc[d] + bufs_v[b, j, pl.ds(LANES * d, LANES)]
                         for d in range(NVEC))
        return inner

    zero = jnp.zeros((LANES,), jnp.float32)
    inv_seq = jnp.float32(1.0 / SEQ)

    def body(g, carry):
        for p in range(2):
            row = 2 * g + p
            acc = (zero,) * NVEC
            for h in range(2):
                b = 2 * p + h
                pltpu.make_async_copy(
                    table_hbm.at[idx_slice(row, h)],
                    bufs_v.at[b].at[pl.ds(0, CH[h])], sems[b]).wait()
                acc = lax.fori_loop(0, CH[h], make_inner(b), acc, unroll=4)

                @pl.when(g < NG - 1)
                def _():
                    pltpu.make_async_copy(
                        table_hbm.at[idx_slice(row + 2, h)],
                        bufs_v.at[b].at[pl.ds(0, CH[h])], sems[b]).start()

            for d in range(NVEC):
                out_v[row, pl.ds(LANES * d, LANES)] = acc[d] * inv_seq
        return carry

    lax.fori_loop(0, NG, body, 0)

    pltpu.sync_copy(out_v, out_hbm.at[pl.ds(wid * ROWS_W, ROWS_W)])


_sc_pool = functools.partial(
    pl.kernel,
    out_type=jax.ShapeDtypeStruct((BATCH, EMBED), jnp.float32),
    mesh=plsc.VectorSubcoreMesh(core_axis_name="c", subcore_axis_name="s",
                                num_cores=NC, num_subcores=NS),
    scratch_types=[
        pltpu.VMEM((ROWS_W, 208), jnp.int32),
        pltpu.VMEM((NBUF, CHMAX, EMBED), jnp.float32),
        pltpu.VMEM((ROWS_W, EMBED), jnp.float32),
        pltpu.SemaphoreType.DMA,
        pltpu.SemaphoreType.DMA,
        pltpu.SemaphoreType.DMA,
        pltpu.SemaphoreType.DMA,
    ],
    compiler_params=pltpu.CompilerParams(use_tc_tiling_on_sc=False),
)(_sc_pool_body)


VCH = 16384
NBLK = (VOCAB + VCH - 1) // VCH


HALF = VCH // 2
VPAD = NBLK * VCH


def _tc_repack_body(tT_ref, o_ref):
    x = tT_ref[...]
    o_ref[:, 0:EMBED] = x[:, 0:HALF].T
    o_ref[:, EMBED:2 * EMBED] = x[:, HALF:VCH].T


_tc_repack = pl.pallas_call(
    _tc_repack_body,
    grid=(NBLK,),
    in_specs=[pl.BlockSpec((EMBED, VCH), lambda i: (0, i))],
    out_specs=pl.BlockSpec((HALF, 2 * EMBED), lambda i: (i, 0)),
    out_shape=jax.ShapeDtypeStruct((NBLK * HALF, 2 * EMBED), jnp.float32),
)


def _mlp_body(x_ref, W1_ref, b1_ref, gamma_ref, beta_ref, W2_ref, b2_ref,
              o_ref):
    x = x_ref[...]
    h = jnp.dot(x, W1_ref[...], preferred_element_type=jnp.float32,
                precision=lax.Precision.HIGHEST) + b1_ref[...]
    mu = jnp.mean(h, axis=0, keepdims=True)
    dev = h - mu
    var = jnp.mean(dev * dev, axis=0, keepdims=True)
    h = dev * lax.rsqrt(var + 1e-5) * gamma_ref[...] + beta_ref[...]
    h = jnp.maximum(h, 0.0)
    logits = jnp.dot(h, W2_ref[...], preferred_element_type=jnp.float32,
                     precision=lax.Precision.HIGHEST) + b2_ref[...]
    m = jnp.max(logits, axis=1, keepdims=True)
    e = jnp.exp(logits - m)
    o_ref[...] = e / jnp.sum(e, axis=1, keepdims=True)


_mlp = pl.pallas_call(
    _mlp_body,
    out_shape=jax.ShapeDtypeStruct((BATCH, CLASSES), jnp.float32),
)


def kernel(tokens, table, W1, b1, gamma, beta, W2, b2):
    packed = _tc_repack(table.T)
    pooled = _sc_pool(tokens, packed.reshape(VPAD, EMBED))
    return _mlp(pooled, W1, b1.reshape(1, HIDDEN), gamma.reshape(1, HIDDEN),
                beta.reshape(1, HIDDEN), W2, b2.reshape(1, CLASSES))


# speedup vs baseline: 2.0942x; 1.0448x over previous
import functools

import jax
import jax.numpy as jnp
from jax import lax
from jax.experimental import pallas as pl
from jax.experimental.pallas import tpu as pltpu
from jax.experimental.pallas import tpu_sc as plsc

VOCAB = 1000000
EMBED = 64
HIDDEN = 256
CLASSES = 100
BATCH = 4096
SEQ = 200

NC = 2
NS = 16
NW = NC * NS
LANES = 16

CH = (104, 96)
CHOFF = (0, 104)
CHMAX = 104
VSH = 15
ROWS_W = BATCH // NW
NBUF = 4
NG = ROWS_W // 2
NVEC = EMBED // LANES


def _sc_pool_body(tokens_hbm, table_hbm, out_hbm, idx_v, bufs_v, out_v,
                  s0, s1, s2, s3):
    sems = (s0, s1, s2, s3)
    wid = lax.axis_index("s") * NC + lax.axis_index("c")

    pltpu.sync_copy(tokens_hbm.at[pl.ds(wid * ROWS_W, ROWS_W)],
                    idx_v.at[:, pl.ds(0, SEQ)])

    def remap(r_, carry):
        for k in range(13):
            t = idx_v[r_, pl.ds(LANES * k, LANES)]
            row = (((t >> VSH) << VSH) + ((t & (VCH // 2 - 1)) << 1)
                   + ((t >> (VSH - 1)) & 1))
            idx_v[r_, pl.ds(LANES * k, LANES)] = row
        return carry

    lax.fori_loop(0, ROWS_W, remap, 0)

    def idx_slice(row, h):
        return idx_v.at[row, pl.ds(CHOFF[h], CH[h])]

    for p in range(2):
        for h in range(2):
            b = 2 * p + h
            pltpu.make_async_copy(
                table_hbm.at[idx_slice(p, h)],
                bufs_v.at[b].at[pl.ds(0, CH[h])], sems[b]).start()

    def make_inner(b):
        def inner(j, acc):
            return tuple(acc[d] + bufs_v[b, j, pl.ds(LANES * d, LANES)]
                         for d in range(NVEC))
        return inner

    zero = jnp.zeros((LANES,), jnp.float32)
    inv_seq = jnp.float32(1.0 / SEQ)

    def body(g, carry):
        for p in range(2):
            row = 2 * g + p
            acc = (zero,) * NVEC
            for h in range(2):
                b = 2 * p + h
                pltpu.make_async_copy(
                    table_hbm.at[idx_slice(row, h)],
                    bufs_v.at[b].at[pl.ds(0, CH[h])], sems[b]).wait()
                acc = lax.fori_loop(0, CH[h], make_inner(b), acc, unroll=4)

                @pl.when(g < NG - 1)
                def _():
                    pltpu.make_async_copy(
                        table_hbm.at[idx_slice(row + 2, h)],
                        bufs_v.at[b].at[pl.ds(0, CH[h])], sems[b]).start()

            for d in range(NVEC):
                out_v[row, pl.ds(LANES * d, LANES)] = acc[d] * inv_seq
        return carry

    lax.fori_loop(0, NG, body, 0)

    pltpu.sync_copy(out_v, out_hbm.at[pl.ds(wid * ROWS_W, ROWS_W)])


_sc_pool = functools.partial(
    pl.kernel,
    out_type=jax.ShapeDtypeStruct((BATCH, EMBED), jnp.float32),
    mesh=plsc.VectorSubcoreMesh(core_axis_name="c", subcore_axis_name="s",
                                num_cores=NC, num_subcores=NS),
    scratch_types=[
        pltpu.VMEM((ROWS_W, 208), jnp.int32),
        pltpu.VMEM((NBUF, CHMAX, EMBED), jnp.float32),
        pltpu.VMEM((ROWS_W, EMBED), jnp.float32),
        pltpu.SemaphoreType.DMA,
        pltpu.SemaphoreType.DMA,
        pltpu.SemaphoreType.DMA,
        pltpu.SemaphoreType.DMA,
    ],
    compiler_params=pltpu.CompilerParams(use_tc_tiling_on_sc=False),
)(_sc_pool_body)


VCH = 32768
NBLK = (VOCAB + VCH - 1) // VCH


HALF = VCH // 2
VPAD = NBLK * VCH


def _tc_repack_body(tT_ref, o_ref):
    x = tT_ref[...]
    o_ref[:, 0:EMBED] = x[:, 0:HALF].T
    o_ref[:, EMBED:2 * EMBED] = x[:, HALF:VCH].T


_tc_repack = pl.pallas_call(
    _tc_repack_body,
    grid=(NBLK,),
    in_specs=[pl.BlockSpec((EMBED, VCH), lambda i: (0, i))],
    out_specs=pl.BlockSpec((HALF, 2 * EMBED), lambda i: (i, 0)),
    out_shape=jax.ShapeDtypeStruct((NBLK * HALF, 2 * EMBED), jnp.float32),
)


def _mlp_body(x_ref, W1_ref, b1_ref, gamma_ref, beta_ref, W2_ref, b2_ref,
              o_ref):
    x = x_ref[...]
    h = jnp.dot(x, W1_ref[...], preferred_element_type=jnp.float32,
                precision=lax.Precision.HIGHEST) + b1_ref[...]
    mu = jnp.mean(h, axis=0, keepdims=True)
    dev = h - mu
    var = jnp.mean(dev * dev, axis=0, keepdims=True)
    h = dev * lax.rsqrt(var + 1e-5) * gamma_ref[...] + beta_ref[...]
    h = jnp.maximum(h, 0.0)
    logits = jnp.dot(h, W2_ref[...], preferred_element_type=jnp.float32,
                     precision=lax.Precision.HIGHEST) + b2_ref[...]
    m = jnp.max(logits, axis=1, keepdims=True)
    e = jnp.exp(logits - m)
    o_ref[...] = e / jnp.sum(e, axis=1, keepdims=True)


_mlp = pl.pallas_call(
    _mlp_body,
    out_shape=jax.ShapeDtypeStruct((BATCH, CLASSES), jnp.float32),
)


def kernel(tokens, table, W1, b1, gamma, beta, W2, b2):
    packed = _tc_repack(table.T)
    pooled = _sc_pool(tokens, packed.reshape(VPAD, EMBED))
    return _mlp(pooled, W1, b1.reshape(1, HIDDEN), gamma.reshape(1, HIDDEN),
                beta.reshape(1, HIDDEN), W2, b2.reshape(1, CLASSES))
